# baseline (speedup 1.0000x reference)
import functools

import jax
import jax.numpy as jnp
from jax import lax
from jax.experimental import pallas as pl
from jax.experimental.pallas import tpu as pltpu
from jax.experimental.pallas import tpu_sc as plsc

NW = 32
NBUF = 8
NGATHER = 4


def _build_gather(n_b: int, n_l: int, d: int):
    rows_per_w = n_b // NW
    n_chunk = rows_per_w

    mesh = plsc.VectorSubcoreMesh(core_axis_name="c", subcore_axis_name="s")

    @functools.partial(
        pl.kernel,
        out_type=jax.ShapeDtypeStruct((n_b, n_l, d), jnp.float32),
        mesh=mesh,
        scratch_types=[
            pltpu.VMEM((rows_per_w, n_l), jnp.int32),
            pltpu.VMEM((NBUF, n_l, d), jnp.float32),
            pltpu.SemaphoreType.DMA,
        ] + [pltpu.SemaphoreType.DMA] * NBUF,
        compiler_params=pltpu.CompilerParams(use_tc_tiling_on_sc=False),
    )
    def gather_kernel(idx_hbm, table_hbm, out_hbm, idx_v, rows_v, gsem,
                      os0, os1, os2, os3, os4, os5, os6, os7):
        osems = (os0, os1, os2, os3, os4, os5, os6, os7)
        wid = lax.axis_index("s") * 2 + lax.axis_index("c")
        row0 = wid * rows_per_w
        pltpu.sync_copy(idx_hbm.at[pl.ds(row0, rows_per_w)], idx_v)

        def wait_gather(j, b):
            pltpu.make_async_copy(
                table_hbm.at[idx_v.at[j]], rows_v.at[b], gsem
            ).wait()

        def fire_gather(j, b):
            pltpu.async_copy(table_hbm.at[idx_v.at[j]], rows_v.at[b], gsem)

        def fire_out(j, b):
            pltpu.async_copy(rows_v.at[b], out_hbm.at[row0 + j], osems[b])

        def wait_out(j, b):
            pltpu.make_async_copy(
                rows_v.at[b], out_hbm.at[row0 + j], osems[b]
            ).wait()

        for b in range(NGATHER):
            fire_gather(b, b)

        for j in range(NBUF - NGATHER):
            wait_gather(j, j % NBUF)
            fire_out(j, j % NBUF)
            fire_gather(j + NGATHER, (j + NGATHER) % NBUF)
        for j in range(NBUF - NGATHER, NBUF):
            wait_gather(j, j % NBUF)
            fire_out(j, j % NBUF)
            wait_out(j - NGATHER, (j + NGATHER) % NBUF)
            fire_gather(j + NGATHER, (j + NGATHER) % NBUF)

        @pl.loop(1, (n_chunk - NGATHER) // NBUF)
        def _grp(g):
            for b in range(NBUF):
                j = g * NBUF + b
                wait_gather(j, b)
                fire_out(j, b)
                b2 = (b + NGATHER) % NBUF
                wait_out(j - NGATHER, b2)
                fire_gather(j + NGATHER, b2)

        tail0 = ((n_chunk - NGATHER) // NBUF) * NBUF
        for j in range(tail0, n_chunk - NGATHER):
            b = j % NBUF
            wait_gather(j, b)
            fire_out(j, b)
            b2 = (b + NGATHER) % NBUF
            wait_out(j - NGATHER, b2)
            fire_gather(j + NGATHER, b2)
        for j in range(n_chunk - NGATHER, n_chunk):
            b = j % NBUF
            wait_gather(j, b)
            fire_out(j, b)
        for j in range(n_chunk - NBUF, n_chunk):
            wait_out(j, j % NBUF)

    return gather_kernel


def kernel(x, table):
    b, l = x.shape
    v, d = table.shape
    return _build_gather(b, l, d)(x, table)


# speedup vs baseline: 1.0108x; 1.0108x over previous
import functools

import jax
import jax.numpy as jnp
from jax import lax
from jax.experimental import pallas as pl
from jax.experimental.pallas import tpu as pltpu
from jax.experimental.pallas import tpu_sc as plsc

NW = 32
CHUNK = 128
NBUF = 8
NGATHER = 4


def _build_gather(n_total: int, d: int):
    b_per_w = n_total // NW
    n_chunk = b_per_w // CHUNK

    mesh = plsc.VectorSubcoreMesh(core_axis_name="c", subcore_axis_name="s")

    @functools.partial(
        pl.kernel,
        out_type=jax.ShapeDtypeStruct((n_total, d), jnp.float32),
        mesh=mesh,
        scratch_types=[
            pltpu.VMEM((n_chunk, CHUNK), jnp.int32),
            pltpu.VMEM((NBUF, CHUNK, d), jnp.float32),
            pltpu.SemaphoreType.DMA,
        ] + [pltpu.SemaphoreType.DMA] * NBUF,
        compiler_params=pltpu.CompilerParams(use_tc_tiling_on_sc=False),
    )
    def gather_kernel(idx_hbm, table_hbm, out_hbm, idx_v, rows_v, gsem,
                      *osems):
        wid = lax.axis_index("s") * 2 + lax.axis_index("c")
        base = wid * b_per_w
        pltpu.sync_copy(idx_hbm.at[wid], idx_v)

        def wait_gather(j, b):
            pltpu.make_async_copy(
                table_hbm.at[idx_v.at[j]], rows_v.at[b], gsem
            ).wait()

        def fire_gather(j, b):
            pltpu.async_copy(table_hbm.at[idx_v.at[j]], rows_v.at[b], gsem)

        def fire_out(j, b):
            pltpu.async_copy(
                rows_v.at[b], out_hbm.at[pl.ds(base + j * CHUNK, CHUNK)],
                osems[b],
            )

        def wait_out(j, b):
            pltpu.make_async_copy(
                rows_v.at[b], out_hbm.at[pl.ds(base + j * CHUNK, CHUNK)],
                osems[b],
            ).wait()

        for b in range(NGATHER):
            fire_gather(b, b)

        for j in range(NBUF - NGATHER):
            wait_gather(j, j % NBUF)
            fire_out(j, j % NBUF)
            fire_gather(j + NGATHER, (j + NGATHER) % NBUF)
        for j in range(NBUF - NGATHER, NBUF):
            wait_gather(j, j % NBUF)
            fire_out(j, j % NBUF)
            wait_out(j - NGATHER, (j + NGATHER) % NBUF)
            fire_gather(j + NGATHER, (j + NGATHER) % NBUF)

        @pl.loop(1, (n_chunk - NGATHER) // NBUF)
        def _grp(g):
            for b in range(NBUF):
                j = g * NBUF + b
                wait_gather(j, b)
                fire_out(j, b)
                b2 = (b + NGATHER) % NBUF
                wait_out(j - NGATHER, b2)
                fire_gather(j + NGATHER, b2)

        tail0 = ((n_chunk - NGATHER) // NBUF) * NBUF
        for j in range(tail0, n_chunk - NGATHER):
            b = j % NBUF
            wait_gather(j, b)
            fire_out(j, b)
            b2 = (b + NGATHER) % NBUF
            wait_out(j - NGATHER, b2)
            fire_gather(j + NGATHER, b2)
        for j in range(n_chunk - NGATHER, n_chunk):
            b = j % NBUF
            wait_gather(j, b)
            fire_out(j, b)
        for j in range(n_chunk - NBUF, n_chunk):
            wait_out(j, j % NBUF)

    return gather_kernel


def kernel(x, table):
    b, l = x.shape
    v, d = table.shape
    n_total = b * l
    idx = x.reshape(NW, n_total // (NW * CHUNK), CHUNK)
    out = _build_gather(n_total, d)(idx, table)
    return out.reshape(b, l, d)


# speedup vs baseline: 1.0121x; 1.0012x over previous
import functools

import jax
import jax.numpy as jnp
from jax import lax
from jax.experimental import pallas as pl
from jax.experimental.pallas import tpu as pltpu
from jax.experimental.pallas import tpu_sc as plsc

NW = 32
CHUNK = 256
NBUF = 6
NGATHER = 3


def _build_gather(n_total: int, d: int):
    b_per_w = n_total // NW
    n_chunk = b_per_w // CHUNK

    mesh = plsc.VectorSubcoreMesh(core_axis_name="c", subcore_axis_name="s")

    @functools.partial(
        pl.kernel,
        out_type=jax.ShapeDtypeStruct((n_total, d), jnp.float32),
        mesh=mesh,
        scratch_types=[
            pltpu.VMEM((n_chunk, CHUNK), jnp.int32),
            pltpu.VMEM((NBUF, CHUNK, d), jnp.float32),
            pltpu.SemaphoreType.DMA,
        ] + [pltpu.SemaphoreType.DMA] * NBUF,
        compiler_params=pltpu.CompilerParams(use_tc_tiling_on_sc=False),
    )
    def gather_kernel(idx_hbm, table_hbm, out_hbm, idx_v, rows_v, gsem,
                      *osems):
        wid = lax.axis_index("s") * 2 + lax.axis_index("c")
        base = wid * b_per_w
        pltpu.sync_copy(idx_hbm.at[wid], idx_v)

        def wait_gather(j, b):
            pltpu.make_async_copy(
                table_hbm.at[idx_v.at[j]], rows_v.at[b], gsem
            ).wait()

        def fire_gather(j, b):
            pltpu.async_copy(table_hbm.at[idx_v.at[j]], rows_v.at[b], gsem)

        def fire_out(j, b):
            pltpu.async_copy(
                rows_v.at[b], out_hbm.at[pl.ds(base + j * CHUNK, CHUNK)],
                osems[b],
            )

        def wait_out(j, b):
            pltpu.make_async_copy(
                rows_v.at[b], out_hbm.at[pl.ds(base + j * CHUNK, CHUNK)],
                osems[b],
            ).wait()

        for b in range(NGATHER):
            fire_gather(b, b)

        for j in range(NBUF - NGATHER):
            wait_gather(j, j % NBUF)
            fire_out(j, j % NBUF)
            fire_gather(j + NGATHER, (j + NGATHER) % NBUF)
        for j in range(NBUF - NGATHER, NBUF):
            wait_gather(j, j % NBUF)
            fire_out(j, j % NBUF)
            wait_out(j - NGATHER, (j + NGATHER) % NBUF)
            fire_gather(j + NGATHER, (j + NGATHER) % NBUF)

        @pl.loop(1, (n_chunk - NGATHER) // NBUF)
        def _grp(g):
            for b in range(NBUF):
                j = g * NBUF + b
                wait_gather(j, b)
                fire_out(j, b)
                b2 = (b + NGATHER) % NBUF
                wait_out(j - NGATHER, b2)
                fire_gather(j + NGATHER, b2)

        tail0 = ((n_chunk - NGATHER) // NBUF) * NBUF
        for j in range(tail0, n_chunk - NGATHER):
            b = j % NBUF
            wait_gather(j, b)
            fire_out(j, b)
            b2 = (b + NGATHER) % NBUF
            wait_out(j - NGATHER, b2)
            fire_gather(j + NGATHER, b2)
        for j in range(n_chunk - NGATHER, n_chunk):
            b = j % NBUF
            wait_gather(j, b)
            fire_out(j, b)
        for j in range(n_chunk - NBUF, n_chunk):
            wait_out(j, j % NBUF)

    return gather_kernel


def kernel(x, table):
    b, l = x.shape
    v, d = table.shape
    n_total = b * l
    idx = x.reshape(NW, n_total // (NW * CHUNK), CHUNK)
    out = _build_gather(n_total, d)(idx, table)
    return out.reshape(b, l, d)
